# baseline (speedup 1.0000x reference)
import functools

import jax
import jax.numpy as jnp
from jax import lax
from jax.experimental import pallas as pl
from jax.experimental.pallas import tpu as pltpu
from jax.experimental.pallas import tpu_sc as plsc

_NC = 2
_NS = 16
_LANES = 16

_K = 16
_NBUF = 3


def _sc_pos_embed(x2d, peX, peY):
    R, D = x2d.shape
    sqn, dh = peX.shape
    nw = _NC * _NS
    rows_per_w = R // nw
    nch = rows_per_w // _K
    groups_per_w = rows_per_w // sqn
    ch_per_group = sqn // _K
    rows_per_tile_a = sqn // _NS
    mesh = plsc.VectorSubcoreMesh(
        core_axis_name="c", subcore_axis_name="s",
        num_cores=_NC, num_subcores=_NS,
    )

    @functools.partial(
        pl.kernel,
        out_type=jax.ShapeDtypeStruct((R, D), jnp.float32),
        mesh=mesh,
        scratch_types=(
            [
                pltpu.HBM((_NC, sqn, D), jnp.float32),
                pltpu.HBM((_NC, sqn, D), jnp.float32),
                pltpu.VMEM((dh,), jnp.float32),
                pltpu.VMEM((D,), jnp.float32),
                pltpu.VMEM((groups_per_w, D), jnp.float32),
                pltpu.VMEM((sqn, D), jnp.float32),
            ]
            + [pltpu.VMEM((_K, D), jnp.float32) for _ in range(_NBUF)]
            + [pltpu.SemaphoreType.DMA for _ in range(2 * _NBUF)]
        ),
        compiler_params=pltpu.CompilerParams(needs_layout_passes=False),
    )
    def body(x_hbm, peX_hbm, peY_hbm, o_hbm, peXi_hbm, peYi_hbm,
             src_v, row_v, pex_l, pey_l, *rest):
        bufs = rest[:_NBUF]
        sin = rest[_NBUF:2 * _NBUF]
        sout = rest[2 * _NBUF:]
        cid = lax.axis_index("c")
        sid = lax.axis_index("s")
        wid = sid * _NC + cid
        base = wid * rows_per_w

        lane = lax.iota(jnp.int32, _LANES)
        parity = lane % 2
        zero = jnp.zeros((_LANES,), jnp.float32)

        for src_hbm, tbl_hbm, off in (
            (peX_hbm, peXi_hbm, 0),
            (peY_hbm, peYi_hbm, 1),
        ):
            for j in range(rows_per_tile_a):
                r = sid * rows_per_tile_a + j
                pltpu.sync_copy(src_hbm.at[r], src_v)
                for k in range(D // _LANES):
                    idx = (k * _LANES + lane) // 2
                    v = plsc.load_gather(src_v, [idx])
                    v = jnp.where(parity == off, v, zero)
                    row_v[pl.ds(k * _LANES, _LANES)] = v
                pltpu.sync_copy(row_v, tbl_hbm.at[cid, r])
        plsc.subcore_barrier()

        workers_per_batch = (sqn * sqn) // rows_per_w
        g0 = (wid % workers_per_batch) * groups_per_w
        pltpu.sync_copy(peXi_hbm.at[cid, pl.ds(g0, groups_per_w)], pex_l)
        pltpu.sync_copy(peYi_hbm.at[cid], pey_l)

        def start_in(g):
            return pltpu.async_copy(
                x_hbm.at[pl.ds(base + g * _K, _K)], bufs[g % _NBUF],
                sin[g % _NBUF])

        def compute(g):
            buf = bufs[g % _NBUF]
            gx = g // ch_per_group
            y0 = (g * _K) % sqn

            def cb_body(cb, carry):
                o = cb * _LANES
                pex_reg = pex_l[gx, pl.ds(o, _LANES)]
                for j in range(_K):
                    buf[j, pl.ds(o, _LANES)] = (
                        buf[j, pl.ds(o, _LANES)]
                        + pex_reg
                        + pey_l[y0 + j, pl.ds(o, _LANES)]
                    )
                return carry

            lax.fori_loop(0, D // _LANES, cb_body, 0)

        def start_out(g):
            return pltpu.async_copy(
                bufs[g % _NBUF], o_hbm.at[pl.ds(base + g * _K, _K)],
                sout[g % _NBUF])

        hin = [None] * nch
        hout = [None] * nch
        for g in range(nch):
            if g >= _NBUF:
                hout[g - _NBUF].wait()
            hin[g] = start_in(g)
            if g >= 1:
                hin[g - 1].wait()
                compute(g - 1)
                hout[g - 1] = start_out(g - 1)
        hin[nch - 1].wait()
        compute(nch - 1)
        hout[nch - 1] = start_out(nch - 1)
        for g in range(nch - _NBUF, nch):
            hout[g].wait()

    return body(x2d, peX, peY)


def kernel(x, peX, peY):
    B, N, D = x.shape
    out = _sc_pos_embed(x.reshape(B * N, D), peX, peY)
    return out.reshape(B, N, D)


# speedup vs baseline: 1.2117x; 1.2117x over previous
import functools

import jax
import jax.numpy as jnp
from jax import lax
from jax.experimental import pallas as pl
from jax.experimental.pallas import tpu as pltpu
from jax.experimental.pallas import tpu_sc as plsc

_NC = 2
_NS = 16
_LANES = 16

_K = 16
_NBUF = 3


def _sc_pos_embed(x2d, peX, peY):
    R, D = x2d.shape
    sqn, dh = peX.shape
    nw = _NC * _NS
    rows_per_w = R // nw
    nch = rows_per_w // _K
    groups_per_w = rows_per_w // sqn
    ch_per_group = sqn // _K
    rows_per_tile_a = sqn // _NS
    mesh = plsc.VectorSubcoreMesh(
        core_axis_name="c", subcore_axis_name="s",
        num_cores=_NC, num_subcores=_NS,
    )

    @functools.partial(
        pl.kernel,
        out_type=jax.ShapeDtypeStruct((R, D), jnp.float32),
        mesh=mesh,
        scratch_types=(
            [
                pltpu.HBM((_NC, sqn, D), jnp.float32),
                pltpu.HBM((_NC, sqn, D), jnp.float32),
                pltpu.VMEM((dh,), jnp.float32),
                pltpu.VMEM((D,), jnp.float32),
                pltpu.VMEM((groups_per_w, D), jnp.float32),
                pltpu.VMEM((sqn, D), jnp.float32),
                pltpu.VMEM((rows_per_w // _K, _LANES), jnp.int32),
            ]
            + [pltpu.VMEM((_K, D), jnp.float32) for _ in range(_NBUF)]
            + [pltpu.SemaphoreType.DMA for _ in range(2 * _NBUF)]
        ),
        compiler_params=pltpu.CompilerParams(needs_layout_passes=False),
    )
    def body(x_hbm, peX_hbm, peY_hbm, o_hbm, peXi_hbm, peYi_hbm,
             src_v, row_v, pex_l, pey_l, idx_all, *rest):
        bufs = rest[:_NBUF]
        sin = rest[_NBUF:2 * _NBUF]
        sout = rest[2 * _NBUF:]
        cid = lax.axis_index("c")
        sid = lax.axis_index("s")
        wid = sid * _NC + cid
        base = wid * rows_per_w

        lane = lax.iota(jnp.int32, _LANES)
        parity = lane % 2
        zero = jnp.zeros((_LANES,), jnp.float32)

        for src_hbm, tbl_hbm, off in (
            (peX_hbm, peXi_hbm, 0),
            (peY_hbm, peYi_hbm, 1),
        ):
            for j in range(rows_per_tile_a):
                r = sid * rows_per_tile_a + j
                pltpu.sync_copy(src_hbm.at[r], src_v)
                for k in range(D // _LANES):
                    idx = (k * _LANES + lane) // 2
                    v = plsc.load_gather(src_v, [idx])
                    v = jnp.where(parity == off, v, zero)
                    row_v[pl.ds(k * _LANES, _LANES)] = v
                pltpu.sync_copy(row_v, tbl_hbm.at[cid, r])
        plsc.subcore_barrier()

        workers_per_batch = (sqn * sqn) // rows_per_w
        g0 = (wid % workers_per_batch) * groups_per_w
        pltpu.sync_copy(peXi_hbm.at[cid, pl.ds(g0, groups_per_w)], pex_l)
        pltpu.sync_copy(peYi_hbm.at[cid], pey_l)
        for g in range(nch):
            idx_all[g, :] = base + g * _K + lane

        def fill(g):
            buf = bufs[g % _NBUF]
            gx = g // ch_per_group
            y0 = (g * _K) % sqn

            def cb_body(cb, carry):
                o = cb * _LANES
                pex_reg = pex_l[gx, pl.ds(o, _LANES)]
                for j in range(_K):
                    buf[j, pl.ds(o, _LANES)] = (
                        pex_reg + pey_l[y0 + j, pl.ds(o, _LANES)]
                    )
                return carry

            lax.fori_loop(0, D // _LANES, cb_body, 0)

        def start_xadd(g):
            return pltpu.async_copy(
                x_hbm.at[idx_all.at[g]], bufs[g % _NBUF],
                sin[g % _NBUF], add=True)

        def start_out(g):
            return pltpu.async_copy(
                bufs[g % _NBUF], o_hbm.at[pl.ds(base + g * _K, _K)],
                sout[g % _NBUF])

        hx = [None] * nch
        hout = [None] * nch
        for g in range(nch):
            if g >= _NBUF:
                hout[g - _NBUF].wait()
            fill(g)
            hx[g] = start_xadd(g)
            if g >= 1:
                hx[g - 1].wait()
                hout[g - 1] = start_out(g - 1)
        hx[nch - 1].wait()
        hout[nch - 1] = start_out(nch - 1)
        for g in range(nch - _NBUF, nch):
            hout[g].wait()

    return body(x2d, peX, peY)


def kernel(x, peX, peY):
    B, N, D = x.shape
    out = _sc_pos_embed(x.reshape(B * N, D), peX, peY)
    return out.reshape(B, N, D)


# speedup vs baseline: 1.5664x; 1.2927x over previous
import functools

import jax
import jax.numpy as jnp
from jax import lax
from jax.experimental import pallas as pl
from jax.experimental.pallas import tpu as pltpu
from jax.experimental.pallas import tpu_sc as plsc

_NC = 2
_NS = 16
_LANES = 16


def _sc_build_tables(peX, peY):
    sqn, dh = peX.shape
    D = 2 * dh
    nw = _NC * _NS
    rows_per_w = (2 * sqn) // nw
    mesh = plsc.VectorSubcoreMesh(
        core_axis_name="c", subcore_axis_name="s",
        num_cores=_NC, num_subcores=_NS,
    )

    @functools.partial(
        pl.kernel,
        out_type=[
            jax.ShapeDtypeStruct((sqn, D), jnp.float32),
            jax.ShapeDtypeStruct((sqn, D), jnp.float32),
        ],
        mesh=mesh,
        scratch_types=(
            [pltpu.VMEM((dh,), jnp.float32) for _ in range(rows_per_w)]
            + [pltpu.VMEM((D,), jnp.float32) for _ in range(rows_per_w)]
            + [pltpu.SemaphoreType.DMA, pltpu.SemaphoreType.DMA]
        ),
        compiler_params=pltpu.CompilerParams(needs_layout_passes=False),
    )
    def build(peX_hbm, peY_hbm, peXi_hbm, peYi_hbm, *scratch):
        srcs = scratch[:rows_per_w]
        rows = scratch[rows_per_w:2 * rows_per_w]
        s_in, s_out = scratch[2 * rows_per_w:]
        wid = lax.axis_index("s") * _NC + lax.axis_index("c")
        zero = jnp.zeros((_LANES,), jnp.float32)
        lane = lax.iota(jnp.int32, _LANES)
        parity = lane % 2
        half = rows_per_w // 2
        tabs = ((peX_hbm, peXi_hbm, 0), (peY_hbm, peYi_hbm, 1))
        hin = []
        for t, (src_hbm, dst_hbm, off) in enumerate(tabs):
            for j in range(half):
                r = wid * half + j
                hin.append(pltpu.async_copy(
                    src_hbm.at[r], srcs[t * half + j], s_in))
        for h in hin:
            h.wait()
        hout = []
        for t, (src_hbm, dst_hbm, off) in enumerate(tabs):
            for j in range(half):
                r = wid * half + j
                src_v = srcs[t * half + j]
                row_v = rows[t * half + j]
                for k in range(D // _LANES):
                    idx = (k * _LANES + lane) // 2
                    v = plsc.load_gather(src_v, [idx])
                    v = jnp.where(parity == off, v, zero)
                    row_v[pl.ds(k * _LANES, _LANES)] = v
                hout.append(pltpu.async_copy(row_v, dst_hbm.at[r], s_out))
        for h in hout:
            h.wait()

    return build(peX, peY)


def _add_body(x_ref, pex_ref, pey_ref, o_ref):
    o_ref[...] = (
        x_ref[...]
        + pex_ref[0][None, None, :, :]
        + pey_ref[...][None, None, :, :]
    )


def kernel(x, peX, peY):
    B, N, D = x.shape
    sqn = peX.shape[0]
    peXi, peYi = _sc_build_tables(peX, peY)
    xr = x.reshape(B, sqn, sqn, D)
    out = pl.pallas_call(
        _add_body,
        grid=(sqn,),
        in_specs=[
            pl.BlockSpec((B, 1, sqn, D), lambda g: (0, g, 0, 0)),
            pl.BlockSpec((1, 1, D), lambda g: (g, 0, 0)),
            pl.BlockSpec((sqn, D), lambda g: (0, 0)),
        ],
        out_specs=pl.BlockSpec((B, 1, sqn, D), lambda g: (0, g, 0, 0)),
        out_shape=jax.ShapeDtypeStruct((B, sqn, sqn, D), x.dtype),
    )(xr, peXi.reshape(sqn, 1, D), peYi)
    return out.reshape(B, N, D)


# speedup vs baseline: 1.5871x; 1.0132x over previous
import functools

import jax
import jax.numpy as jnp
from jax import lax
from jax.experimental import pallas as pl
from jax.experimental.pallas import tpu as pltpu
from jax.experimental.pallas import tpu_sc as plsc

_NC = 2
_NS = 16
_LANES = 16


def _sc_build_tables(peX, peY):
    sqn, dh = peX.shape
    D = 2 * dh
    nw = _NC * _NS
    rows_per_w = (2 * sqn) // nw
    mesh = plsc.VectorSubcoreMesh(
        core_axis_name="c", subcore_axis_name="s",
        num_cores=_NC, num_subcores=_NS,
    )

    @functools.partial(
        pl.kernel,
        out_type=[
            jax.ShapeDtypeStruct((sqn, D), jnp.float32),
            jax.ShapeDtypeStruct((sqn, D), jnp.float32),
        ],
        mesh=mesh,
        scratch_types=(
            [pltpu.VMEM((dh,), jnp.float32) for _ in range(rows_per_w)]
            + [pltpu.VMEM((D,), jnp.float32) for _ in range(rows_per_w)]
            + [pltpu.SemaphoreType.DMA, pltpu.SemaphoreType.DMA]
        ),
        compiler_params=pltpu.CompilerParams(needs_layout_passes=False),
    )
    def build(peX_hbm, peY_hbm, peXi_hbm, peYi_hbm, *scratch):
        srcs = scratch[:rows_per_w]
        rows = scratch[rows_per_w:2 * rows_per_w]
        s_in, s_out = scratch[2 * rows_per_w:]
        wid = lax.axis_index("s") * _NC + lax.axis_index("c")
        zero = jnp.zeros((_LANES,), jnp.float32)
        lane = lax.iota(jnp.int32, _LANES)
        parity = lane % 2
        half = rows_per_w // 2
        tabs = ((peX_hbm, peXi_hbm, 0), (peY_hbm, peYi_hbm, 1))
        hin = []
        for t, (src_hbm, dst_hbm, off) in enumerate(tabs):
            for j in range(half):
                r = wid * half + j
                hin.append(pltpu.async_copy(
                    src_hbm.at[r], srcs[t * half + j], s_in))
        for h in hin:
            h.wait()
        hout = []
        for t, (src_hbm, dst_hbm, off) in enumerate(tabs):
            for j in range(half):
                r = wid * half + j
                src_v = srcs[t * half + j]
                row_v = rows[t * half + j]

                def chunk(k, carry, src_v=src_v, row_v=row_v, off=off):
                    idx = (k * _LANES + lane) // 2
                    v = plsc.load_gather(src_v, [idx])
                    v = jnp.where(parity == off, v, zero)
                    row_v[pl.ds(k * _LANES, _LANES)] = v
                    return carry

                lax.fori_loop(0, D // _LANES, chunk, 0)
                hout.append(pltpu.async_copy(row_v, dst_hbm.at[r], s_out))
        for h in hout:
            h.wait()

    return build(peX, peY)


def _add_body(x_ref, pex_ref, pey_ref, o_ref):
    o_ref[...] = (
        x_ref[...]
        + pex_ref[0][None, None, :, :]
        + pey_ref[...][None, None, :, :]
    )


def kernel(x, peX, peY):
    B, N, D = x.shape
    sqn = peX.shape[0]
    peXi, peYi = _sc_build_tables(peX, peY)
    xr = x.reshape(B, sqn, sqn, D)
    out = pl.pallas_call(
        _add_body,
        grid=(sqn,),
        in_specs=[
            pl.BlockSpec((B, 1, sqn, D), lambda g: (0, g, 0, 0)),
            pl.BlockSpec((1, 1, D), lambda g: (g, 0, 0)),
            pl.BlockSpec((sqn, D), lambda g: (0, 0)),
        ],
        out_specs=pl.BlockSpec((B, 1, sqn, D), lambda g: (0, g, 0, 0)),
        out_shape=jax.ShapeDtypeStruct((B, sqn, sqn, D), x.dtype),
    )(xr, peXi.reshape(sqn, 1, D), peYi)
    return out.reshape(B, N, D)
